# baseline (speedup 1.0000x reference)
import functools

import jax
import jax.numpy as jnp
from jax import lax
from jax.experimental import pallas as pl
from jax.experimental.pallas import tpu as pltpu
from jax.experimental.pallas import tpu_sc as plsc

UNITS, NEURONS, IN_DIM, OUT_DIM, BATCH = 16, 512, 256, 256, 32
_L = 16


def _lr_sc_body(x_hbm, alr_hbm, out_hbm, xv, av, ov):
    b = lax.axis_index("s") * 2 + lax.axis_index("c")
    pltpu.sync_copy(x_hbm.at[b], xv)
    pltpu.sync_copy(alr_hbm, av)
    lanes = lax.iota(jnp.int32, _L)

    def unit_logit(h, logits):
        base = h * IN_DIM
        part = jnp.zeros((_L,), jnp.float32)
        for j in range(IN_DIM // _L):
            sl = pl.ds(base + j * _L, _L)
            part = part + xv[sl] * av[sl]
        return jnp.where(lanes == h, jnp.sum(part), logits)

    logits = lax.fori_loop(0, UNITS, unit_logit, jnp.zeros((_L,), jnp.float32))
    m = jnp.max(logits)
    e = jnp.exp(logits - m)
    ov[...] = e / jnp.sum(e)
    pltpu.sync_copy(ov, out_hbm.at[b])


def _lr_sc_body2(x_hbm, alr_hbm, out_hbm, xv, av, ov):
    s_id = lax.axis_index("s")
    pltpu.sync_copy(x_hbm.at[pl.ds(s_id * 2, 2)], xv)
    pltpu.sync_copy(alr_hbm, av)
    lanes = lax.iota(jnp.int32, _L)
    for r in range(2):
        def unit_logit(h, logits):
            base = h * IN_DIM
            part = jnp.zeros((_L,), jnp.float32)
            for j in range(IN_DIM // _L):
                sl = pl.ds(base + j * _L, _L)
                part = part + xv[r, sl] * av[sl]
            return jnp.where(lanes == h, jnp.sum(part), logits)

        logits = lax.fori_loop(0, UNITS, unit_logit,
                               jnp.zeros((_L,), jnp.float32))
        m = jnp.max(logits)
        e = jnp.exp(logits - m)
        ov[r, :] = e / jnp.sum(e)
    pltpu.sync_copy(ov, out_hbm.at[pl.ds(s_id * 2, 2)])


def _lr_sparsecore(X, adaptive_lr, temperature):
    x_flat = X.reshape(BATCH, UNITS * IN_DIM)
    alr_flat = (adaptive_lr[:, :, 0] / temperature[0]).reshape(UNITS * IN_DIM)
    mesh = plsc.VectorSubcoreMesh(core_axis_name="c", subcore_axis_name="s",
                                  num_cores=1)
    run = pl.kernel(
        _lr_sc_body2,
        out_type=jax.ShapeDtypeStruct((BATCH, UNITS), jnp.float32),
        mesh=mesh,
        scratch_types=[
            pltpu.VMEM((2, UNITS * IN_DIM), jnp.float32),
            pltpu.VMEM((UNITS * IN_DIM,), jnp.float32),
            pltpu.VMEM((2, UNITS), jnp.float32),
        ],
        compiler_params=pltpu.CompilerParams(needs_layout_passes=False),
    )
    return run(x_flat, alr_flat)


_NH = NEURONS // 2


def _unit_body(x_ref, s_ref, sh_ref, w_ref, win_ref, b_ref, wout_ref, sr_ref,
               lr_ref, ns_ref, out_ref):
    j = pl.program_id(1)
    x = x_ref[...]
    s = s_ref[...]
    feed = jnp.dot(x, win_ref[0], preferred_element_type=jnp.float32)
    echo = jnp.dot(s, w_ref[0], preferred_element_type=jnp.float32)
    echo = echo * sr_ref[0, 0, 0] + b_ref[0]
    lr = lr_ref[0]
    ns = (1.0 - lr) * sh_ref[...] + lr * jnp.tanh(feed + echo)
    ns_ref[...] = ns
    contrib = jnp.dot(ns, wout_ref[0], preferred_element_type=jnp.float32)

    @pl.when(j == 0)
    def _():
        out_ref[...] = contrib

    @pl.when(j == 1)
    def _():
        out_ref[...] += contrib


def _units_tensorcore(Xf, Sf, W, Win, bias, Wout, sr, lrU):
    unit3 = lambda h, j: (h, 0, 0)
    colh = lambda h, j: (0, h)
    half = lambda h, j: (h, 0, j)
    return pl.pallas_call(
        _unit_body,
        grid=(UNITS, 2),
        in_specs=[
            pl.BlockSpec((BATCH, IN_DIM), colh),
            pl.BlockSpec((BATCH, NEURONS), colh),
            pl.BlockSpec((BATCH, _NH), lambda h, j: (0, 2 * h + j)),
            pl.BlockSpec((1, NEURONS, _NH), half),
            pl.BlockSpec((1, IN_DIM, _NH), half),
            pl.BlockSpec((1, 1, _NH), half),
            pl.BlockSpec((1, _NH, OUT_DIM), lambda h, j: (h, j, 0)),
            pl.BlockSpec((1, 1, 1), unit3),
            pl.BlockSpec((1, BATCH, 1), unit3),
        ],
        out_specs=[
            pl.BlockSpec((BATCH, _NH), lambda h, j: (0, 2 * h + j)),
            pl.BlockSpec((BATCH, OUT_DIM), colh),
        ],
        out_shape=[
            jax.ShapeDtypeStruct((BATCH, UNITS * NEURONS), jnp.float32),
            jax.ShapeDtypeStruct((BATCH, UNITS * OUT_DIM), jnp.float32),
        ],
    )(Xf, Sf, Sf, W, Win, bias, Wout, sr, lrU)


def kernel(X, state, W, Win, bias, Wout, sr, adaptive_lr, temperature,
           w_pos, win_pos, xw_pos, xwin_pos):
    lr = _lr_sparsecore(X, adaptive_lr, temperature)
    lrU = jnp.transpose(lr, (1, 0))[:, :, None]
    Xf = X.reshape(BATCH, UNITS * IN_DIM)
    Sf = state.reshape(BATCH, UNITS * NEURONS)
    ns_f, out_f = _units_tensorcore(Xf, Sf, W, Win, bias, Wout, sr, lrU)
    return (ns_f.reshape(BATCH, UNITS, NEURONS),
            out_f.reshape(BATCH, UNITS, OUT_DIM))


# speedup vs baseline: 1.2053x; 1.2053x over previous
import jax
import jax.numpy as jnp
from jax import lax
from jax.experimental import pallas as pl
from jax.experimental.pallas import tpu as pltpu
from jax.experimental.pallas import tpu_sc as plsc

UNITS, NEURONS, IN_DIM, OUT_DIM, BATCH = 16, 512, 256, 256, 32
_L = 16


def _lr_sc_body2(x_hbm, alr_hbm, out_hbm, xv, av, ov):
    s_id = lax.axis_index("s")
    pltpu.sync_copy(x_hbm.at[pl.ds(s_id * 2, 2)], xv)
    pltpu.sync_copy(alr_hbm, av)
    lanes = lax.iota(jnp.int32, _L)
    for r in range(2):
        def unit_logit(h, logits):
            base = h * IN_DIM
            part = jnp.zeros((_L,), jnp.float32)
            for j in range(IN_DIM // _L):
                sl = pl.ds(base + j * _L, _L)
                part = part + xv[r, sl] * av[sl]
            return jnp.where(lanes == h, jnp.sum(part), logits)

        logits = lax.fori_loop(0, UNITS, unit_logit,
                               jnp.zeros((_L,), jnp.float32))
        m = jnp.max(logits)
        e = jnp.exp(logits - m)
        ov[r, :] = e / jnp.sum(e)
    pltpu.sync_copy(ov, out_hbm.at[pl.ds(s_id * 2, 2)])


def _lr_sparsecore(X, adaptive_lr, temperature):
    x_flat = X.reshape(BATCH, UNITS * IN_DIM)
    alr_flat = (adaptive_lr[:, :, 0] / temperature[0]).reshape(UNITS * IN_DIM)
    mesh = plsc.VectorSubcoreMesh(core_axis_name="c", subcore_axis_name="s",
                                  num_cores=1)
    run = pl.kernel(
        _lr_sc_body2,
        out_type=jax.ShapeDtypeStruct((BATCH, UNITS), jnp.float32),
        mesh=mesh,
        scratch_types=[
            pltpu.VMEM((2, UNITS * IN_DIM), jnp.float32),
            pltpu.VMEM((UNITS * IN_DIM,), jnp.float32),
            pltpu.VMEM((2, UNITS), jnp.float32),
        ],
        compiler_params=pltpu.CompilerParams(needs_layout_passes=False),
    )
    return run(x_flat, alr_flat)


def _unit_body(x_ref, s_ref, w_ref, win_ref, b_ref, wout_ref, sr_ref, lr_ref,
               ns_ref, out_ref):
    x = x_ref[...]
    s = s_ref[...]
    feed = jnp.dot(x, win_ref[0], preferred_element_type=jnp.float32)
    echo = jnp.dot(s, w_ref[0], preferred_element_type=jnp.float32)
    echo = echo * sr_ref[0, 0, 0] + b_ref[0]
    lr = lr_ref[0]
    ns = (1.0 - lr) * s + lr * jnp.tanh(feed + echo)
    ns_ref[...] = ns
    out_ref[...] = jnp.dot(ns, wout_ref[0], preferred_element_type=jnp.float32)


def _units_tensorcore(Xf, Sf, W, Win, bias, Wout, sr, lrU):
    unit3 = lambda h: (h, 0, 0)
    col = lambda h: (0, h)
    return pl.pallas_call(
        _unit_body,
        grid=(UNITS,),
        in_specs=[
            pl.BlockSpec((BATCH, IN_DIM), col),
            pl.BlockSpec((BATCH, NEURONS), col),
            pl.BlockSpec((1, NEURONS, NEURONS), unit3),
            pl.BlockSpec((1, IN_DIM, NEURONS), unit3),
            pl.BlockSpec((1, 1, NEURONS), unit3),
            pl.BlockSpec((1, NEURONS, OUT_DIM), unit3),
            pl.BlockSpec((1, 1, 1), unit3),
            pl.BlockSpec((1, BATCH, 1), unit3),
        ],
        out_specs=[
            pl.BlockSpec((BATCH, NEURONS), col),
            pl.BlockSpec((BATCH, OUT_DIM), col),
        ],
        out_shape=[
            jax.ShapeDtypeStruct((BATCH, UNITS * NEURONS), jnp.float32),
            jax.ShapeDtypeStruct((BATCH, UNITS * OUT_DIM), jnp.float32),
        ],
    )(Xf, Sf, W, Win, bias, Wout, sr, lrU)


def kernel(X, state, W, Win, bias, Wout, sr, adaptive_lr, temperature,
           w_pos, win_pos, xw_pos, xwin_pos):
    lr = _lr_sparsecore(X, adaptive_lr, temperature)
    lrU = jnp.transpose(lr, (1, 0))[:, :, None]
    Xf = X.reshape(BATCH, UNITS * IN_DIM)
    Sf = state.reshape(BATCH, UNITS * NEURONS)
    ns_f, out_f = _units_tensorcore(Xf, Sf, W, Win, bias, Wout, sr, lrU)
    return (ns_f.reshape(BATCH, UNITS, NEURONS),
            out_f.reshape(BATCH, UNITS, OUT_DIM))
